# baseline (speedup 1.0000x reference)
import functools

import jax
import jax.numpy as jnp
from jax import lax
from jax.experimental import pallas as pl
from jax.experimental.pallas import tpu as pltpu
from jax.experimental.pallas import tpu_sc as plsc

EPS = 1e-12
H = 128
NLANE = 16
NCHUNK = H // NLANE


def _lane_allreduce_sum(x):
    dn = lax.GatherDimensionNumbers(
        offset_dims=(), collapsed_slice_dims=(0,), start_index_map=(0,))
    lane = lax.iota(jnp.int32, NLANE)
    for d in (8, 4, 2, 1):
        idx = (lane + d) & (NLANE - 1)
        perm = lax.gather(x, idx[:, None], dn, slice_sizes=(1,),
                          mode=lax.GatherScatterMode.PROMISE_IN_BOUNDS)
        x = x + perm
    return x


def _ln_kernel_body(L, rows_per_worker, num_cores,
                    ids, vec, wemb, pemb, gamma, beta, out,
                    idx_v, x_v, pos_v, gam_v, bet_v, sem_g, sem_w):
    Lp1 = L + 1
    half_l = L // 2
    nb = rows_per_worker
    wid = lax.axis_index("s") * num_cores + lax.axis_index("c")
    base = wid * nb

    pltpu.sync_copy(pemb, pos_v)
    pltpu.sync_copy(gamma, gam_v)
    pltpu.sync_copy(beta, bet_v)
    g = [gam_v[pl.ds(k * NLANE, NLANE)] for k in range(NCHUNK)]
    bt = [bet_v[pl.ds(k * NLANE, NLANE)] for k in range(NCHUNK)]

    def stage_and_fire(row, buf):
        pltpu.sync_copy(ids.at[row], idx_v.at[buf])
        pltpu.sync_copy(vec.at[row], x_v.at[buf, pl.ds(0, 1)])
        pltpu.async_copy(wemb.at[idx_v.at[buf, 0]],
                         x_v.at[buf, pl.ds(1, half_l)], sem_g.at[buf])
        pltpu.async_copy(wemb.at[idx_v.at[buf, 1]],
                         x_v.at[buf, pl.ds(1 + half_l, half_l)], sem_g.at[buf])

    def wait_gather(buf):
        pltpu.make_async_copy(wemb.at[idx_v.at[buf, 0]],
                              x_v.at[buf, pl.ds(1, half_l)], sem_g.at[buf]).wait()
        pltpu.make_async_copy(wemb.at[idx_v.at[buf, 1]],
                              x_v.at[buf, pl.ds(1 + half_l, half_l)], sem_g.at[buf]).wait()

    def wait_writeback(row, buf):
        pltpu.make_async_copy(x_v.at[buf], out.at[row], sem_w.at[buf]).wait()

    stage_and_fire(base, 0)

    def do_batch_row(j, carry):
        buf = j % 3
        nxt = (j + 1) % 3

        @pl.when(j + 1 < nb)
        def _fire_next():
            @pl.when(j >= 2)
            def _drain_wb():
                wait_writeback(base + j - 2, nxt)
            stage_and_fire(base + j + 1, nxt)

        wait_gather(buf)

        @plsc.parallel_loop(0, Lp1, step=1, unroll=6)
        def ln_row(t):
            xs = [x_v[buf, t, pl.ds(k * NLANE, NLANE)] + pos_v[t, pl.ds(k * NLANE, NLANE)]
                  for k in range(NCHUNK)]
            s = ((xs[0] + xs[1]) + (xs[2] + xs[3])) + ((xs[4] + xs[5]) + (xs[6] + xs[7]))
            sq = [x * x for x in xs]
            ss = ((sq[0] + sq[1]) + (sq[2] + sq[3])) + ((sq[4] + sq[5]) + (sq[6] + sq[7]))
            mean_v = _lane_allreduce_sum(s) * (1.0 / H)
            var_v = _lane_allreduce_sum(ss) * (1.0 / H) - mean_v * mean_v
            v = var_v + EPS
            i = lax.bitcast_convert_type(v, jnp.int32)
            i = 0x5F3759DF - (i >> 1)
            y = lax.bitcast_convert_type(i, jnp.float32)
            vh = v * 0.5
            y = y * (1.5 - vh * y * y)
            y = y * (1.5 - vh * y * y)
            for k in range(NCHUNK):
                xn = (xs[k] - mean_v) * y
                x_v[buf, t, pl.ds(k * NLANE, NLANE)] = xn * g[k] + bt[k]

        pltpu.async_copy(x_v.at[buf], out.at[base + j], sem_w.at[buf])
        return carry

    lax.fori_loop(0, nb, do_batch_row, 0)
    for i in range(3):
        row = nb - 3 + i
        wait_writeback(base + row, row % 3)


def kernel(input_ids, vectors, word_emb, pos_emb, ln_gamma, ln_beta):
    B, L = input_ids.shape
    info = plsc.get_sparse_core_info()
    num_workers = info.num_cores * info.num_subcores
    rows_per_worker = B // num_workers
    ids = input_ids.astype(jnp.int32).reshape(B, 2, L // 2)
    vec3 = vectors.reshape(B, 1, H)
    pos_used = pos_emb[1:L + 2]

    mesh = plsc.VectorSubcoreMesh(core_axis_name="c", subcore_axis_name="s")
    body = functools.partial(_ln_kernel_body, L, rows_per_worker, info.num_cores)
    run = pl.kernel(
        body,
        out_type=jax.ShapeDtypeStruct((B, L + 1, H), jnp.float32),
        mesh=mesh,
        scratch_types=[
            pltpu.VMEM((3, 2, L // 2), jnp.int32),
            pltpu.VMEM((3, L + 1, H), jnp.float32),
            pltpu.VMEM((L + 1, H), jnp.float32),
            pltpu.VMEM((H,), jnp.float32),
            pltpu.VMEM((H,), jnp.float32),
            pltpu.SemaphoreType.DMA((3,)),
            pltpu.SemaphoreType.DMA((3,)),
        ],
    )
    return run(ids, vec3, word_emb, pos_used, ln_gamma, ln_beta)


# speedup vs baseline: 1.4929x; 1.4929x over previous
import functools

import jax
import jax.numpy as jnp
from jax import lax
from jax.experimental import pallas as pl
from jax.experimental.pallas import tpu as pltpu
from jax.experimental.pallas import tpu_sc as plsc

EPS = 1e-12
H = 128
NLANE = 16
NCHUNK = H // NLANE


def _lane_allreduce_sum(x):
    dn = lax.GatherDimensionNumbers(
        offset_dims=(), collapsed_slice_dims=(0,), start_index_map=(0,))
    lane = lax.iota(jnp.int32, NLANE)
    for d in (8, 4, 2, 1):
        idx = (lane + d) & (NLANE - 1)
        perm = lax.gather(x, idx[:, None], dn, slice_sizes=(1,),
                          mode=lax.GatherScatterMode.PROMISE_IN_BOUNDS)
        x = x + perm
    return x


def _ln_kernel_body(L, rows_per_worker, num_cores,
                    ids, vec, wemb, pemb, gamma, beta, out,
                    idx_v, x_v, pos_v, gam_v, bet_v, sem_g, sem_w):
    Lp1 = L + 1
    half_l = L // 2
    nb = rows_per_worker
    wid = lax.axis_index("s") * num_cores + lax.axis_index("c")
    base = wid * nb

    pltpu.sync_copy(pemb, pos_v)
    pltpu.sync_copy(gamma, gam_v)
    pltpu.sync_copy(beta, bet_v)
    g = [gam_v[pl.ds(k * NLANE, NLANE)] for k in range(NCHUNK)]
    bt = [bet_v[pl.ds(k * NLANE, NLANE)] for k in range(NCHUNK)]

    def stage_and_fire(row, buf):
        pltpu.sync_copy(ids.at[row], idx_v.at[buf])
        pltpu.sync_copy(vec.at[row], x_v.at[buf, pl.ds(0, 1)])
        pltpu.async_copy(wemb.at[idx_v.at[buf, 0]],
                         x_v.at[buf, pl.ds(1, half_l)], sem_g.at[buf])
        pltpu.async_copy(wemb.at[idx_v.at[buf, 1]],
                         x_v.at[buf, pl.ds(1 + half_l, half_l)], sem_g.at[buf])

    def wait_gather(buf):
        pltpu.make_async_copy(wemb.at[idx_v.at[buf, 0]],
                              x_v.at[buf, pl.ds(1, half_l)], sem_g.at[buf]).wait()
        pltpu.make_async_copy(wemb.at[idx_v.at[buf, 1]],
                              x_v.at[buf, pl.ds(1 + half_l, half_l)], sem_g.at[buf]).wait()

    def wait_writeback(row, buf):
        pltpu.make_async_copy(x_v.at[buf], out.at[row], sem_w.at[buf]).wait()

    stage_and_fire(base, 0)

    def do_batch_row(j, carry):
        buf = j % 3
        nxt = (j + 1) % 3

        @pl.when(j + 1 < nb)
        def _fire_next():
            @pl.when(j >= 2)
            def _drain_wb():
                wait_writeback(base + j - 2, nxt)
            stage_and_fire(base + j + 1, nxt)

        wait_gather(buf)

        @plsc.parallel_loop(0, Lp1, step=1, unroll=3)
        def ln_row(t):
            xs = [x_v[buf, t, pl.ds(k * NLANE, NLANE)] + pos_v[t, pl.ds(k * NLANE, NLANE)]
                  for k in range(NCHUNK)]
            s = ((xs[0] + xs[1]) + (xs[2] + xs[3])) + ((xs[4] + xs[5]) + (xs[6] + xs[7]))
            sq = [x * x for x in xs]
            ss = ((sq[0] + sq[1]) + (sq[2] + sq[3])) + ((sq[4] + sq[5]) + (sq[6] + sq[7]))
            mean_v = _lane_allreduce_sum(s) * (1.0 / H)
            var_v = _lane_allreduce_sum(ss) * (1.0 / H) - mean_v * mean_v
            v = var_v + EPS
            i = lax.bitcast_convert_type(v, jnp.int32)
            i = 0x5F3759DF - (i >> 1)
            y = lax.bitcast_convert_type(i, jnp.float32)
            vh = v * 0.5
            y = y * (1.5 - vh * y * y)
            y = y * (1.5 - vh * y * y)
            for k in range(NCHUNK):
                xn = (xs[k] - mean_v) * y
                x_v[buf, t, pl.ds(k * NLANE, NLANE)] = xn * g[k] + bt[k]

        pltpu.async_copy(x_v.at[buf], out.at[base + j], sem_w.at[buf])
        return carry

    lax.fori_loop(0, nb, do_batch_row, 0)
    for i in range(3):
        row = nb - 3 + i
        wait_writeback(base + row, row % 3)


def kernel(input_ids, vectors, word_emb, pos_emb, ln_gamma, ln_beta):
    B, L = input_ids.shape
    info = plsc.get_sparse_core_info()
    num_workers = info.num_cores * info.num_subcores
    rows_per_worker = B // num_workers
    ids = input_ids.astype(jnp.int32).reshape(B, 2, L // 2)
    vec3 = vectors.reshape(B, 1, H)
    pos_used = pos_emb[1:L + 2]

    mesh = plsc.VectorSubcoreMesh(core_axis_name="c", subcore_axis_name="s")
    body = functools.partial(_ln_kernel_body, L, rows_per_worker, info.num_cores)
    run = pl.kernel(
        body,
        out_type=jax.ShapeDtypeStruct((B, L + 1, H), jnp.float32),
        mesh=mesh,
        scratch_types=[
            pltpu.VMEM((3, 2, L // 2), jnp.int32),
            pltpu.VMEM((3, L + 1, H), jnp.float32),
            pltpu.VMEM((L + 1, H), jnp.float32),
            pltpu.VMEM((H,), jnp.float32),
            pltpu.VMEM((H,), jnp.float32),
            pltpu.SemaphoreType.DMA((3,)),
            pltpu.SemaphoreType.DMA((3,)),
        ],
    )
    return run(ids, vec3, word_emb, pos_used, ln_gamma, ln_beta)
